# baseline (speedup 1.0000x reference)
import functools

import jax
import jax.numpy as jnp
from jax import lax
from jax.experimental import pallas as pl
from jax.experimental.pallas import tpu as pltpu
from jax.experimental.pallas import tpu_sc as plsc

H = 128
NH = 8
HD = 16
NF = 64
IN_DIM = 4
PI = 3.14159265358979323846

NW = 32
CH = 40
NPAD = 10240
NPK = NPAD // 16


def _ln(x, w, b, eps=1e-5):
    mu = jnp.mean(x, axis=-1, keepdims=True)
    var = jnp.mean((x - mu) ** 2, axis=-1, keepdims=True)
    return (x - mu) / jnp.sqrt(var + eps) * w + b


def _floatmod(x, y):
    r = lax.rem(x, y)
    return jnp.where((r != 0) & ((r < 0) != (y < 0)), r + y, r)


def _sincos2pi(p):
    k = jnp.floor(2.0 * p + 0.5)
    r = p - 0.5 * k
    r2 = r * r
    s = r * (6.2831852 + r2 * (-41.341665 + r2 * (81.60163164
        + r2 * (-76.5646677 + r2 * 39.6528002))))
    c = 1.0 + r2 * (-19.7392086 + r2 * (64.9393554
        + r2 * (-85.4540081 + r2 * (60.1527725 + r2 * -25.0430879))))
    half = 0.5 * k
    sgn = jnp.where(half == jnp.floor(half), 1.0, -1.0)
    return sgn * s, sgn * c


def _prologue_body(T, x_ref, pos_ref, head_ref,
                   psw, psb, pdw, pdb, qw, qb, kw, vw, vb,
                   f3, w1c3, w1s3, w1l3, b13, ln3w, ln3b, w23, b23,
                   src_out, dst_out, t3_out):
    i = pl.program_id(0)
    tv = (lax.broadcasted_iota(jnp.int32, (104, 1), 0) - (T - 1)
          ).astype(jnp.float32)
    xi3 = tv * f3[...]
    h3 = (jnp.dot(jnp.cos(xi3), w1c3[...], preferred_element_type=jnp.float32)
          + jnp.dot(jnp.sin(xi3), w1s3[...], preferred_element_type=jnp.float32)
          + tv * w1l3[...] + b13[...])
    h3 = _ln(h3, ln3w[...], ln3b[...])
    h3 = jnp.maximum(h3, 0.0)
    t3_out[...] = jnp.dot(h3, w23[...], preferred_element_type=jnp.float32) \
        + b23[...]
    x = x_ref[...]
    B = x.shape[0]
    xs = _ln(x, psw[...], psb[...])
    xd = _ln(x, pdw[...], pdb[...])
    q = jnp.dot(xd, qw[...], preferred_element_type=jnp.float32) + qb[...]
    k = jnp.dot(xs, kw[...], preferred_element_type=jnp.float32)
    v = jnp.dot(xs, vw[...], preferred_element_type=jnp.float32) + vb[...]
    px = pos_ref[:, 0:1]
    py = pos_ref[:, 1:2]
    hd = head_ref[...]
    row = lax.broadcasted_iota(jnp.int32, (B, 1), 0) + i * B
    tf = lax.rem(row, T).astype(jnp.float32)
    m16 = lax.rem(row, 16).astype(jnp.float32)
    z1 = jnp.zeros((B, 1), jnp.float32)
    feat = jnp.concatenate(
        [px, py, hd, tf, jnp.cos(hd), jnp.sin(hd), m16, z1], axis=1)

    def pack2(a, b):
        au = lax.bitcast_convert_type(a.astype(jnp.bfloat16), jnp.uint16)
        bu = lax.bitcast_convert_type(b.astype(jnp.bfloat16), jnp.uint16)
        word = (au.astype(jnp.uint32) << 16) | bu.astype(jnp.uint32)
        return lax.bitcast_convert_type(word, jnp.float32)

    kv = pack2(k, v)
    qp = pack2(q[:, 0:64], q[:, 64:128])
    src_out[...] = jnp.concatenate(
        [kv, feat, jnp.zeros((B, 120), jnp.float32)], axis=1)
    dst_out[...] = jnp.concatenate(
        [qp, feat, jnp.zeros((B, 56), jnp.float32)], axis=1)


def _run_prologue(T, x, pos, head, p):
    N = x.shape[0]
    RB = 2000
    grid = N // RB
    full = lambda a: pl.BlockSpec(a.shape, lambda i: (0,) * a.ndim)
    row = lambda w: pl.BlockSpec((RB, w), lambda i: (i, 0))
    vec = lambda name: p[name].reshape(1, -1)
    i3 = 3
    w13 = p['mlp%d_w1' % i3]
    args3 = [(p['freq_%d' % i3] * (2.0 * PI)).reshape(1, NF),
             w13[0:NF, :], w13[NF:2 * NF, :], w13[2 * NF:, :],
             vec('mlp%d_b1' % i3), vec('mlp%d_lnw' % i3),
             vec('mlp%d_lnb' % i3), p['mlp%d_w2' % i3],
             vec('mlp%d_b2' % i3)]
    return pl.pallas_call(
        functools.partial(_prologue_body, T),
        grid=(grid,),
        in_specs=[row(H), row(2), row(1)] + [full(jnp.zeros((1, H)))] * 4 +
                 [full(jnp.zeros((H, H))), full(jnp.zeros((1, H))),
                  full(jnp.zeros((H, H))), full(jnp.zeros((H, H))),
                  full(jnp.zeros((1, H)))] + [full(a) for a in args3],
        out_specs=[row(2 * H), row(H),
                   pl.BlockSpec((104, H), lambda i: (0, 0))],
        out_shape=[jax.ShapeDtypeStruct((N, 2 * H), jnp.float32),
                   jax.ShapeDtypeStruct((N, H), jnp.float32),
                   jax.ShapeDtypeStruct((104, H), jnp.float32)],
    )(x, pos, head,
      vec('pre_src_w'), vec('pre_src_b'), vec('pre_dst_w'), vec('pre_dst_b'),
      p['q_w'], vec('q_b'), p['k_w'], p['v_w'], vec('v_b'), *args3)


def _run_gather(srct, dstt, sidx, didx):
    E = sidx.shape[0] * sidx.shape[1] * sidx.shape[2]
    kch = sidx.shape[1]
    ew = kch * CH
    mesh = plsc.VectorSubcoreMesh(core_axis_name="c", subcore_axis_name="s")

    @functools.partial(
        pl.kernel, mesh=mesh,
        out_type=[jax.ShapeDtypeStruct((E, 2 * H), jnp.float32),
                  jax.ShapeDtypeStruct((E, H), jnp.float32)],
        scratch_types=[pltpu.VMEM((kch, CH), jnp.int32),
                       pltpu.VMEM((kch, CH), jnp.int32),
                       pltpu.VMEM((CH, 2 * H), jnp.float32),
                       pltpu.VMEM((CH, 2 * H), jnp.float32),
                       pltpu.VMEM((CH, H), jnp.float32),
                       pltpu.VMEM((CH, H), jnp.float32),
                       pltpu.SemaphoreType.DMA,
                       pltpu.SemaphoreType.DMA,
                       pltpu.SemaphoreType.DMA,
                       pltpu.SemaphoreType.DMA],
    )
    def gather_k(srct_hbm, dstt_hbm, sidx_hbm, didx_hbm,
                 sg_out, dg_out, sidx_v, didx_v,
                 s0, s1, d0, d1, g0, g1, w0, w1):
        wid = lax.axis_index("s") * 2 + lax.axis_index("c")
        pltpu.sync_copy(sidx_hbm.at[wid], sidx_v)
        pltpu.sync_copy(didx_hbm.at[wid], didx_v)
        sbuf, dbuf = [s0, s1], [d0, d1]
        gsem, wsem = [g0, g1], [w0, w1]
        gcp, wcp = [None] * kch, [None] * kch
        for j in range(kch):
            b = j % 2
            if j >= 2:
                wcp[j - 2][0].wait()
                wcp[j - 2][1].wait()
            gcp[j] = (pltpu.async_copy(srct_hbm.at[sidx_v.at[j]],
                                       sbuf[b], gsem[b]),
                      pltpu.async_copy(dstt_hbm.at[didx_v.at[j]],
                                       dbuf[b], gsem[b]))
            if j >= 1:
                p = j - 1
                pb = p % 2
                gcp[p][0].wait()
                gcp[p][1].wait()
                base = wid * ew + p * CH
                wcp[p] = (pltpu.async_copy(sbuf[pb],
                                           sg_out.at[pl.ds(base, CH)],
                                           wsem[pb]),
                          pltpu.async_copy(dbuf[pb],
                                           dg_out.at[pl.ds(base, CH)],
                                           wsem[pb]))
        last = kch - 1
        lb = last % 2
        gcp[last][0].wait()
        gcp[last][1].wait()
        base = wid * ew + last * CH
        pltpu.sync_copy(sbuf[lb], sg_out.at[pl.ds(base, CH)])
        pltpu.sync_copy(dbuf[lb], dg_out.at[pl.ds(base, CH)])
        if kch >= 2:
            wcp[kch - 2][0].wait()
            wcp[kch - 2][1].wait()

    return gather_k(srct, dstt, sidx, didx)


def _edge_body(toff, sg, dg, t3, freq2pi, w1c, w1s, w1l, b1, ln1w, ln1b,
               w2, b2, folnw, folnb, fow, fob, prw, prb, krw, vrw, vrb,
               smat, rmat, e1g, e2f, pk_ref, exr_ref):
    kvbits = lax.bitcast_convert_type(sg[:, 0:H], jnp.uint32)
    kg = lax.bitcast_convert_type(kvbits & jnp.uint32(0xFFFF0000),
                                  jnp.float32)
    vg = lax.bitcast_convert_type(kvbits << 16, jnp.float32)
    qbits = lax.bitcast_convert_type(dg[:, 0:64], jnp.uint32)
    qi = jnp.concatenate(
        [lax.bitcast_convert_type(qbits & jnp.uint32(0xFFFF0000),
                                  jnp.float32),
         lax.bitcast_convert_type(qbits << 16, jnp.float32)], axis=1)
    fs = sg[:, H:H + 8]
    fd = dg[:, 64:72]
    rel_x = fs[:, 0:1] - fd[:, 0:1]
    rel_y = fs[:, 1:2] - fd[:, 1:2]
    dist = jnp.sqrt(rel_x * rel_x + rel_y * rel_y)
    chd = fd[:, 4:5]
    shd = fd[:, 5:6]
    cross = chd * rel_y - shd * rel_x
    dot = chd * rel_x + shd * rel_y
    direction = lax.atan2(cross, dot)
    rel_head = _floatmod(fs[:, 2:3] - fd[:, 2:3] + PI, 2.0 * PI) - PI
    time_rel = fs[:, 3:4] - fd[:, 3:4]
    rel = [dist, direction, rel_head]
    onehot = (time_rel + toff ==
              lax.broadcasted_iota(jnp.int32, (1, 104), 1).astype(jnp.float32)
              ).astype(jnp.float32)
    acc = jnp.dot(onehot, t3, preferred_element_type=jnp.float32)
    for i in range(IN_DIM - 1):
        sn, cs = _sincos2pi(rel[i] * freq2pi[i])
        h = (jnp.dot(cs, w1c[i], preferred_element_type=jnp.float32)
             + jnp.dot(sn, w1s[i], preferred_element_type=jnp.float32)
             + rel[i] * w1l[i] + b1[i])
        h = _ln(h, ln1w[i], ln1b[i])
        h = jnp.maximum(h, 0.0)
        h = jnp.dot(h, w2[i], preferred_element_type=jnp.float32) + b2[i]
        acc = h if acc is None else acc + h
    acc = _ln(acc, folnw[...], folnb[...])
    acc = jnp.maximum(acc, 0.0)
    r_emb = jnp.dot(acc, fow[...], preferred_element_type=jnp.float32) + fob[...]
    r = _ln(r_emb, prw[...], prb[...])
    kj = kg + jnp.dot(r, krw[...], preferred_element_type=jnp.float32)
    vj = (vg + jnp.dot(r, vrw[...], preferred_element_type=jnp.float32)
          + vrb[...])
    siml = qi * kj * (HD ** -0.5)
    sim = jnp.dot(siml, smat[...], preferred_element_type=jnp.float32)
    ex = jnp.exp(sim)
    exb = jnp.dot(ex, rmat[...], preferred_element_type=jnp.float32)
    pk_ref[...] = exb * vj
    g16 = (fd[:, 6:7] ==
           lax.broadcasted_iota(jnp.int32, (1, 16), 1).astype(jnp.float32))
    exr_ref[...] = (jnp.dot(g16.astype(jnp.float32), e1g[...],
                            preferred_element_type=jnp.float32) *
                    jnp.dot(ex, e2f[...], preferred_element_type=jnp.float32))


def _edge_kernel_fn(toff, *refs):
    ins = [r[...] for r in refs[:-2]]
    _edge_body(toff, *ins, pk_ref=refs[-2], exr_ref=refs[-1])


def _run_edges(sg, dg, t3, toff, wpack):
    E = sg.shape[0]
    EB = 1280
    grid = E // EB
    full = lambda a: pl.BlockSpec(a.shape, lambda i: (0,) * a.ndim)
    return pl.pallas_call(
        functools.partial(_edge_kernel_fn, toff),
        grid=(grid,),
        in_specs=[pl.BlockSpec((EB, 2 * H), lambda i: (i, 0)),
                  pl.BlockSpec((EB, H), lambda i: (i, 0)),
                  full(t3)] + [full(a) for a in wpack],
        out_specs=[pl.BlockSpec((EB, H), lambda i: (i, 0)),
                   pl.BlockSpec((EB, H), lambda i: (i, 0))],
        out_shape=[jax.ShapeDtypeStruct((E, H), jnp.float32),
                   jax.ShapeDtypeStruct((E, H), jnp.float32)],
    )(sg, dg, t3, *wpack)


def _run_scatter(pk, exr, didx, didx2, zeros):
    E = pk.shape[0]
    kch = didx.shape[1]
    ew = kch * CH
    rpt = NPAD // 16
    rpt2 = NPK // 16
    mesh = plsc.VectorSubcoreMesh(core_axis_name="c", subcore_axis_name="s")

    @functools.partial(
        pl.kernel, mesh=mesh,
        out_type=[jax.ShapeDtypeStruct((2, NPAD, H), jnp.float32),
                  jax.ShapeDtypeStruct((2, NPK, H), jnp.float32)],
        scratch_types=[pltpu.VMEM((kch, CH), jnp.int32),
                       pltpu.VMEM((kch, CH), jnp.int32),
                       pltpu.VMEM((CH, H), jnp.float32),
                       pltpu.VMEM((CH, H), jnp.float32),
                       pltpu.VMEM((CH, H), jnp.float32),
                       pltpu.VMEM((CH, H), jnp.float32),
                       pltpu.VMEM_SHARED((NPAD, H), jnp.float32),
                       pltpu.VMEM_SHARED((NPK, H), jnp.float32),
                       pltpu.SemaphoreType.DMA,
                       pltpu.SemaphoreType.DMA],
    )
    def scatter_k(pk_hbm, exr_hbm, didx_hbm, didx2_hbm, zeros_hbm,
                  out_hbm, out2_hbm, idx_v, idx2_v, p0, p1, x0, x1,
                  acc, acc2, r0, r1):
        c = lax.axis_index("c")
        s = lax.axis_index("s")
        wid = s * 2 + c
        pltpu.sync_copy(didx_hbm.at[wid], idx_v)
        pltpu.sync_copy(didx2_hbm.at[wid], idx2_v)
        pbuf, xbuf, rsem = [p0, p1], [x0, x1], [r0, r1]
        rcp = [None] * kch
        for j in range(min(2, kch)):
            base = wid * ew + j * CH
            rcp[j] = (pltpu.async_copy(pk_hbm.at[pl.ds(base, CH)],
                                       pbuf[j], rsem[j]),
                      pltpu.async_copy(exr_hbm.at[pl.ds(base, CH)],
                                       xbuf[j], rsem[j]))
        pltpu.sync_copy(zeros_hbm.at[pl.ds(s * rpt, rpt)],
                        acc.at[pl.ds(s * rpt, rpt)])
        pltpu.sync_copy(zeros_hbm.at[pl.ds(s * rpt2, rpt2)],
                        acc2.at[pl.ds(s * rpt2, rpt2)])
        plsc.subcore_barrier()
        for j in range(kch):
            b = j % 2
            rcp[j][0].wait()
            rcp[j][1].wait()
            pltpu.sync_copy(pbuf[b], acc.at[idx_v.at[j]], add=True)
            pltpu.sync_copy(xbuf[b], acc2.at[idx2_v.at[j]], add=True)
            if j + 2 < kch:
                base = wid * ew + (j + 2) * CH
                rcp[j + 2] = (pltpu.async_copy(pk_hbm.at[pl.ds(base, CH)],
                                               pbuf[b], rsem[b]),
                              pltpu.async_copy(exr_hbm.at[pl.ds(base, CH)],
                                               xbuf[b], rsem[b]))
        plsc.subcore_barrier()
        pltpu.sync_copy(acc.at[pl.ds(s * rpt, rpt)],
                        out_hbm.at[c, pl.ds(s * rpt, rpt)])
        pltpu.sync_copy(acc2.at[pl.ds(s * rpt2, rpt2)],
                        out2_hbm.at[c, pl.ds(s * rpt2, rpt2)])

    return scatter_k(pk, exr, didx, didx2, zeros)


def _epi_body(npart, x_ref, *refs):
    nums = refs[:npart]
    dens = refs[npart:2 * npart]
    (pdw, pdb, ga, gx, gb, sw, sb, ow, ob,
     postw, postb, ffprew, ffpreb, ffw1, ffb1, ffw2, ffb2,
     ffpostw, ffpostb, rmat, out_ref) = refs[2 * npart:]
    x = x_ref[...]
    num = nums[0][...]
    for r in nums[1:]:
        num = num + r[...]
    den8 = dens[0][...]
    for r in dens[1:]:
        den8 = den8 + r[...]
    den = jnp.dot(den8, rmat[...], preferred_element_type=jnp.float32) + 1e-16
    agg = num / den
    xd = _ln(x, pdw[...], pdb[...])
    glog = (jnp.dot(agg, ga[...], preferred_element_type=jnp.float32)
            + jnp.dot(xd, gx[...], preferred_element_type=jnp.float32)
            + gb[...])
    g = 1.0 / (1.0 + jnp.exp(-glog))
    s = jnp.dot(xd, sw[...], preferred_element_type=jnp.float32) + sb[...]
    upd = agg + g * (s - agg)
    attn_out = jnp.dot(upd, ow[...], preferred_element_type=jnp.float32) + ob[...]
    x1 = x + _ln(attn_out, postw[...], postb[...])
    ffin = _ln(x1, ffprew[...], ffpreb[...])
    hdn = jnp.maximum(
        jnp.dot(ffin, ffw1[...], preferred_element_type=jnp.float32) + ffb1[...],
        0.0)
    h2 = jnp.dot(hdn, ffw2[...], preferred_element_type=jnp.float32) + ffb2[...]
    out_ref[...] = x1 + _ln(h2, ffpostw[...], ffpostb[...])


def _run_epilogue(x, nums, dens, p, rmat):
    N = x.shape[0]
    RB = 2000
    grid = N // RB
    npart = len(nums)
    full = lambda a: pl.BlockSpec(a.shape, lambda i: (0,) * a.ndim)
    row = lambda w: pl.BlockSpec((RB, w), lambda i: (i, 0))
    vec = lambda name: p[name].reshape(1, -1)
    args = [vec('pre_dst_w'), vec('pre_dst_b'),
            p['g_w'][:H, :], p['g_w'][H:, :], vec('g_b'),
            p['s_w'], vec('s_b'), p['o_w'], vec('o_b'),
            vec('post_w'), vec('post_b'), vec('ffpre_w'), vec('ffpre_b'),
            p['ff_w1'], vec('ff_b1'), p['ff_w2'], vec('ff_b2'),
            vec('ffpost_w'), vec('ffpost_b'), rmat]
    return pl.pallas_call(
        functools.partial(_epi_body, npart),
        grid=(grid,),
        in_specs=[row(H)] + [row(H)] * npart + [row(8)] * npart +
                 [full(a) for a in args],
        out_specs=row(H),
        out_shape=jax.ShapeDtypeStruct((N, H), jnp.float32),
    )(x, *nums, *dens, *args)


def _edge_weight_pack(p):
    w1 = jnp.stack([p['mlp%d_w1' % i] for i in range(IN_DIM)])
    pack = [
        jnp.stack([p['freq_%d' % i] for i in range(IN_DIM)]),
        w1[:, 0:NF, :],
        w1[:, NF:2 * NF, :],
        w1[:, 2 * NF, :],
        jnp.stack([p['mlp%d_b1' % i] for i in range(IN_DIM)]),
        jnp.stack([p['mlp%d_lnw' % i] for i in range(IN_DIM)]),
        jnp.stack([p['mlp%d_lnb' % i] for i in range(IN_DIM)]),
        jnp.stack([p['mlp%d_w2' % i] for i in range(IN_DIM)]),
        jnp.stack([p['mlp%d_b2' % i] for i in range(IN_DIM)]),
        p['fe_out_lnw'].reshape(1, H),
        p['fe_out_lnb'].reshape(1, H),
        p['fe_out_w'],
        p['fe_out_b'].reshape(1, H),
        p['pre_r_w'].reshape(1, H),
        p['pre_r_b'].reshape(1, H),
        p['kr_w'],
        p['vr_w'],
        p['vr_b'].reshape(1, H),
    ]
    hid = jnp.arange(H, dtype=jnp.int32) // HD
    smat = (hid[:, None] == jnp.arange(NH)[None, :]).astype(jnp.float32)
    rmat = smat.T
    lane = jnp.arange(H, dtype=jnp.int32)
    e1g = (jnp.arange(16)[:, None] == (lane // 8)[None, :]).astype(jnp.float32)
    e2f = (jnp.arange(NH)[:, None] == (lane % 8)[None, :]).astype(jnp.float32)
    pack += [smat, rmat, e1g, e2f]
    return pack, rmat


def kernel(src_x, src_pos, src_head, edges, params):
    A, M, T, C = src_x.shape
    N = A * M * T
    E = edges.shape[1]
    x = src_x.reshape(N, C)
    pos = src_pos.reshape(N, 2)
    head = src_head.reshape(N, 1)
    edges = edges.astype(jnp.int32)
    kch = E // (NW * CH)
    nch = 5
    splits = [(kch + i) // nch for i in range(nch)]
    bounds = [0]
    for s in splits:
        bounds.append(bounds[-1] + NW * CH * s)

    srct, dstt, t3 = _run_prologue(T, x, pos, head, params)
    wpack, rmat = _edge_weight_pack(params)
    zeros = jnp.zeros((NPAD, H), jnp.float32)
    nums, dens = [], []
    for lo, hi in zip(bounds[:-1], bounds[1:]):
        kc = (hi - lo) // (NW * CH)
        sidx = edges[0, lo:hi].reshape(NW, kc, CH)
        didx = edges[1, lo:hi].reshape(NW, kc, CH)
        sg, dg = _run_gather(srct, dstt, sidx, didx)
        pk, exr = _run_edges(sg, dg, t3, float(T - 1), wpack)
        part, part2 = _run_scatter(pk, exr, didx, didx // 16, zeros)
        den = part2.reshape(2, NPAD, 8)
        nums += [part[0, :N], part[1, :N]]
        dens += [den[0, :N], den[1, :N]]
    out = _run_epilogue(x, nums, dens, params, rmat)
    return out.reshape(A, M, T, C)


# speedup vs baseline: 1.0120x; 1.0120x over previous
import functools

import jax
import jax.numpy as jnp
from jax import lax
from jax.experimental import pallas as pl
from jax.experimental.pallas import tpu as pltpu
from jax.experimental.pallas import tpu_sc as plsc

H = 128
NH = 8
HD = 16
NF = 64
IN_DIM = 4
PI = 3.14159265358979323846

NW = 32
CH = 40
NPAD = 10240
NPK = NPAD // 16


def _ln(x, w, b, eps=1e-5):
    mu = jnp.mean(x, axis=-1, keepdims=True)
    var = jnp.mean((x - mu) ** 2, axis=-1, keepdims=True)
    return (x - mu) / jnp.sqrt(var + eps) * w + b


def _floatmod(x, y):
    r = lax.rem(x, y)
    return jnp.where((r != 0) & ((r < 0) != (y < 0)), r + y, r)


def _sincos2pi(p):
    k = jnp.floor(2.0 * p + 0.5)
    r = p - 0.5 * k
    r2 = r * r
    s = r * (6.2831852 + r2 * (-41.341665 + r2 * (81.60163164
        + r2 * (-76.5646677 + r2 * 39.6528002))))
    c = 1.0 + r2 * (-19.7392086 + r2 * (64.9393554
        + r2 * (-85.4540081 + r2 * (60.1527725 + r2 * -25.0430879))))
    half = 0.5 * k
    sgn = jnp.where(half == jnp.floor(half), 1.0, -1.0)
    return sgn * s, sgn * c


def _prologue_body(T, x_ref, pos_ref, head_ref,
                   psw, psb, pdw, pdb, qw, qb, kw, vw, vb,
                   f3, w1c3, w1s3, w1l3, b13, ln3w, ln3b, w23, b23,
                   src_out, dst_out, t3_out):
    i = pl.program_id(0)
    tv = (lax.broadcasted_iota(jnp.int32, (104, 1), 0) - (T - 1)
          ).astype(jnp.float32)
    xi3 = tv * f3[...]
    h3 = (jnp.dot(jnp.cos(xi3), w1c3[...], preferred_element_type=jnp.float32)
          + jnp.dot(jnp.sin(xi3), w1s3[...], preferred_element_type=jnp.float32)
          + tv * w1l3[...] + b13[...])
    h3 = _ln(h3, ln3w[...], ln3b[...])
    h3 = jnp.maximum(h3, 0.0)
    t3_out[...] = jnp.dot(h3, w23[...], preferred_element_type=jnp.float32) \
        + b23[...]
    x = x_ref[...]
    B = x.shape[0]
    xs = _ln(x, psw[...], psb[...])
    xd = _ln(x, pdw[...], pdb[...])
    q = jnp.dot(xd, qw[...], preferred_element_type=jnp.float32) + qb[...]
    k = jnp.dot(xs, kw[...], preferred_element_type=jnp.float32)
    v = jnp.dot(xs, vw[...], preferred_element_type=jnp.float32) + vb[...]
    px = pos_ref[:, 0:1]
    py = pos_ref[:, 1:2]
    hd = head_ref[...]
    row = lax.broadcasted_iota(jnp.int32, (B, 1), 0) + i * B
    tf = lax.rem(row, T).astype(jnp.float32)
    m16 = lax.rem(row, 16).astype(jnp.float32)
    z1 = jnp.zeros((B, 1), jnp.float32)
    feat = jnp.concatenate(
        [px, py, hd, tf, jnp.cos(hd), jnp.sin(hd), m16, z1], axis=1)

    def pack2(a, b):
        au = lax.bitcast_convert_type(a.astype(jnp.bfloat16), jnp.uint16)
        bu = lax.bitcast_convert_type(b.astype(jnp.bfloat16), jnp.uint16)
        word = (au.astype(jnp.uint32) << 16) | bu.astype(jnp.uint32)
        return lax.bitcast_convert_type(word, jnp.float32)

    kv = pack2(k, v)
    qp = pack2(q[:, 0:64], q[:, 64:128])
    src_out[...] = jnp.concatenate(
        [kv, feat, jnp.zeros((B, 120), jnp.float32)], axis=1)
    dst_out[...] = jnp.concatenate(
        [qp, feat, jnp.zeros((B, 56), jnp.float32)], axis=1)


def _run_prologue(T, x, pos, head, p):
    N = x.shape[0]
    RB = 2000
    grid = N // RB
    full = lambda a: pl.BlockSpec(a.shape, lambda i: (0,) * a.ndim)
    row = lambda w: pl.BlockSpec((RB, w), lambda i: (i, 0))
    vec = lambda name: p[name].reshape(1, -1)
    i3 = 3
    w13 = p['mlp%d_w1' % i3]
    args3 = [(p['freq_%d' % i3] * (2.0 * PI)).reshape(1, NF),
             w13[0:NF, :], w13[NF:2 * NF, :], w13[2 * NF:, :],
             vec('mlp%d_b1' % i3), vec('mlp%d_lnw' % i3),
             vec('mlp%d_lnb' % i3), p['mlp%d_w2' % i3],
             vec('mlp%d_b2' % i3)]
    return pl.pallas_call(
        functools.partial(_prologue_body, T),
        grid=(grid,),
        in_specs=[row(H), row(2), row(1)] + [full(jnp.zeros((1, H)))] * 4 +
                 [full(jnp.zeros((H, H))), full(jnp.zeros((1, H))),
                  full(jnp.zeros((H, H))), full(jnp.zeros((H, H))),
                  full(jnp.zeros((1, H)))] + [full(a) for a in args3],
        out_specs=[row(2 * H), row(H),
                   pl.BlockSpec((104, H), lambda i: (0, 0))],
        out_shape=[jax.ShapeDtypeStruct((N, 2 * H), jnp.float32),
                   jax.ShapeDtypeStruct((N, H), jnp.float32),
                   jax.ShapeDtypeStruct((104, H), jnp.float32)],
    )(x, pos, head,
      vec('pre_src_w'), vec('pre_src_b'), vec('pre_dst_w'), vec('pre_dst_b'),
      p['q_w'], vec('q_b'), p['k_w'], p['v_w'], vec('v_b'), *args3)


def _run_gather(srct, dstt, sidx, didx):
    E = sidx.shape[0] * sidx.shape[1] * sidx.shape[2]
    kch = sidx.shape[1]
    ew = kch * CH
    mesh = plsc.VectorSubcoreMesh(core_axis_name="c", subcore_axis_name="s")

    @functools.partial(
        pl.kernel, mesh=mesh,
        out_type=[jax.ShapeDtypeStruct((E, 2 * H), jnp.float32),
                  jax.ShapeDtypeStruct((E, H), jnp.float32)],
        scratch_types=[pltpu.VMEM((kch, CH), jnp.int32),
                       pltpu.VMEM((kch, CH), jnp.int32),
                       pltpu.VMEM((CH, 2 * H), jnp.float32),
                       pltpu.VMEM((CH, 2 * H), jnp.float32),
                       pltpu.VMEM((CH, H), jnp.float32),
                       pltpu.VMEM((CH, H), jnp.float32),
                       pltpu.SemaphoreType.DMA,
                       pltpu.SemaphoreType.DMA,
                       pltpu.SemaphoreType.DMA,
                       pltpu.SemaphoreType.DMA],
    )
    def gather_k(srct_hbm, dstt_hbm, sidx_hbm, didx_hbm,
                 sg_out, dg_out, sidx_v, didx_v,
                 s0, s1, d0, d1, g0, g1, w0, w1):
        wid = lax.axis_index("s") * 2 + lax.axis_index("c")
        pltpu.sync_copy(sidx_hbm.at[wid], sidx_v)
        pltpu.sync_copy(didx_hbm.at[wid], didx_v)
        sbuf, dbuf = [s0, s1], [d0, d1]
        gsem, wsem = [g0, g1], [w0, w1]
        gcp, wcp = [None] * kch, [None] * kch
        for j in range(kch):
            b = j % 2
            if j >= 2:
                wcp[j - 2][0].wait()
                wcp[j - 2][1].wait()
            gcp[j] = (pltpu.async_copy(srct_hbm.at[sidx_v.at[j]],
                                       sbuf[b], gsem[b]),
                      pltpu.async_copy(dstt_hbm.at[didx_v.at[j]],
                                       dbuf[b], gsem[b]))
            if j >= 1:
                p = j - 1
                pb = p % 2
                gcp[p][0].wait()
                gcp[p][1].wait()
                base = wid * ew + p * CH
                wcp[p] = (pltpu.async_copy(sbuf[pb],
                                           sg_out.at[pl.ds(base, CH)],
                                           wsem[pb]),
                          pltpu.async_copy(dbuf[pb],
                                           dg_out.at[pl.ds(base, CH)],
                                           wsem[pb]))
        last = kch - 1
        lb = last % 2
        gcp[last][0].wait()
        gcp[last][1].wait()
        base = wid * ew + last * CH
        pltpu.sync_copy(sbuf[lb], sg_out.at[pl.ds(base, CH)])
        pltpu.sync_copy(dbuf[lb], dg_out.at[pl.ds(base, CH)])
        if kch >= 2:
            wcp[kch - 2][0].wait()
            wcp[kch - 2][1].wait()

    return gather_k(srct, dstt, sidx, didx)


def _edge_body(toff, sg, dg, t3, freq2pi, w1c, w1s, w1l, b1, ln1w, ln1b,
               w2, b2, folnw, folnb, fow, fob, prw, prb, krw, vrw, vrb,
               smat, rmat, e1g, e2f, pk_ref, exr_ref):
    kvbits = lax.bitcast_convert_type(sg[:, 0:H], jnp.uint32)
    kg = lax.bitcast_convert_type(kvbits & jnp.uint32(0xFFFF0000),
                                  jnp.float32)
    vg = lax.bitcast_convert_type(kvbits << 16, jnp.float32)
    qbits = lax.bitcast_convert_type(dg[:, 0:64], jnp.uint32)
    qi = jnp.concatenate(
        [lax.bitcast_convert_type(qbits & jnp.uint32(0xFFFF0000),
                                  jnp.float32),
         lax.bitcast_convert_type(qbits << 16, jnp.float32)], axis=1)
    fs = sg[:, H:H + 8]
    fd = dg[:, 64:72]
    rel_x = fs[:, 0:1] - fd[:, 0:1]
    rel_y = fs[:, 1:2] - fd[:, 1:2]
    dist = jnp.sqrt(rel_x * rel_x + rel_y * rel_y)
    chd = fd[:, 4:5]
    shd = fd[:, 5:6]
    cross = chd * rel_y - shd * rel_x
    dot = chd * rel_x + shd * rel_y
    direction = lax.atan2(cross, dot)
    rel_head = _floatmod(fs[:, 2:3] - fd[:, 2:3] + PI, 2.0 * PI) - PI
    time_rel = fs[:, 3:4] - fd[:, 3:4]
    rel = [dist, direction, rel_head]
    onehot = (time_rel + toff ==
              lax.broadcasted_iota(jnp.int32, (1, 104), 1).astype(jnp.float32)
              ).astype(jnp.float32)
    acc = jnp.dot(onehot, t3, preferred_element_type=jnp.float32)
    for i in range(IN_DIM - 1):
        sn, cs = _sincos2pi(rel[i] * freq2pi[i])
        h = (jnp.dot(cs, w1c[i], preferred_element_type=jnp.float32)
             + jnp.dot(sn, w1s[i], preferred_element_type=jnp.float32)
             + rel[i] * w1l[i] + b1[i])
        h = _ln(h, ln1w[i], ln1b[i])
        h = jnp.maximum(h, 0.0)
        h = jnp.dot(h, w2[i], preferred_element_type=jnp.float32) + b2[i]
        acc = h if acc is None else acc + h
    acc = _ln(acc, folnw[...], folnb[...])
    acc = jnp.maximum(acc, 0.0)
    r_emb = jnp.dot(acc, fow[...], preferred_element_type=jnp.float32) + fob[...]
    r = _ln(r_emb, prw[...], prb[...])
    kj = kg + jnp.dot(r, krw[...], preferred_element_type=jnp.float32)
    vj = (vg + jnp.dot(r, vrw[...], preferred_element_type=jnp.float32)
          + vrb[...])
    siml = qi * kj * (HD ** -0.5)
    sim = jnp.dot(siml, smat[...], preferred_element_type=jnp.float32)
    ex = jnp.exp(sim)
    exb = jnp.dot(ex, rmat[...], preferred_element_type=jnp.float32)
    pk_ref[...] = exb * vj
    g16 = (fd[:, 6:7] ==
           lax.broadcasted_iota(jnp.int32, (1, 16), 1).astype(jnp.float32))
    exr_ref[...] = (jnp.dot(g16.astype(jnp.float32), e1g[...],
                            preferred_element_type=jnp.float32) *
                    jnp.dot(ex, e2f[...], preferred_element_type=jnp.float32))


def _edge_kernel_fn(toff, *refs):
    ins = [r[...] for r in refs[:-2]]
    _edge_body(toff, *ins, pk_ref=refs[-2], exr_ref=refs[-1])


def _run_edges(sg, dg, t3, toff, wpack):
    E = sg.shape[0]
    EB = 1280
    grid = E // EB
    full = lambda a: pl.BlockSpec(a.shape, lambda i: (0,) * a.ndim)
    return pl.pallas_call(
        functools.partial(_edge_kernel_fn, toff),
        grid=(grid,),
        in_specs=[pl.BlockSpec((EB, 2 * H), lambda i: (i, 0)),
                  pl.BlockSpec((EB, H), lambda i: (i, 0)),
                  full(t3)] + [full(a) for a in wpack],
        out_specs=[pl.BlockSpec((EB, H), lambda i: (i, 0)),
                   pl.BlockSpec((EB, H), lambda i: (i, 0))],
        out_shape=[jax.ShapeDtypeStruct((E, H), jnp.float32),
                   jax.ShapeDtypeStruct((E, H), jnp.float32)],
    )(sg, dg, t3, *wpack)


def _run_scatter(pk, exr, didx, didx2, zeros):
    E = pk.shape[0]
    kch = didx.shape[1]
    ew = kch * CH
    rpt = NPAD // 16
    rpt2 = NPK // 16
    mesh = plsc.VectorSubcoreMesh(core_axis_name="c", subcore_axis_name="s")

    @functools.partial(
        pl.kernel, mesh=mesh,
        out_type=[jax.ShapeDtypeStruct((2, NPAD, H), jnp.float32),
                  jax.ShapeDtypeStruct((2, NPK, H), jnp.float32)],
        scratch_types=[pltpu.VMEM((kch, CH), jnp.int32),
                       pltpu.VMEM((kch, CH), jnp.int32),
                       pltpu.VMEM((CH, H), jnp.float32),
                       pltpu.VMEM((CH, H), jnp.float32),
                       pltpu.VMEM((CH, H), jnp.float32),
                       pltpu.VMEM((CH, H), jnp.float32),
                       pltpu.VMEM_SHARED((NPAD, H), jnp.float32),
                       pltpu.VMEM_SHARED((NPK, H), jnp.float32),
                       pltpu.SemaphoreType.DMA,
                       pltpu.SemaphoreType.DMA],
    )
    def scatter_k(pk_hbm, exr_hbm, didx_hbm, didx2_hbm, zeros_hbm,
                  out_hbm, out2_hbm, idx_v, idx2_v, p0, p1, x0, x1,
                  acc, acc2, r0, r1):
        c = lax.axis_index("c")
        s = lax.axis_index("s")
        wid = s * 2 + c
        pltpu.sync_copy(didx_hbm.at[wid], idx_v)
        pltpu.sync_copy(didx2_hbm.at[wid], idx2_v)
        pbuf, xbuf, rsem = [p0, p1], [x0, x1], [r0, r1]
        rcp = [None] * kch
        for j in range(min(2, kch)):
            base = wid * ew + j * CH
            rcp[j] = (pltpu.async_copy(pk_hbm.at[pl.ds(base, CH)],
                                       pbuf[j], rsem[j]),
                      pltpu.async_copy(exr_hbm.at[pl.ds(base, CH)],
                                       xbuf[j], rsem[j]))
        pltpu.sync_copy(zeros_hbm.at[pl.ds(s * rpt, rpt)],
                        acc.at[pl.ds(s * rpt, rpt)])
        pltpu.sync_copy(zeros_hbm.at[pl.ds(s * rpt2, rpt2)],
                        acc2.at[pl.ds(s * rpt2, rpt2)])
        plsc.subcore_barrier()
        for j in range(kch):
            b = j % 2
            rcp[j][0].wait()
            rcp[j][1].wait()
            pltpu.sync_copy(pbuf[b], acc.at[idx_v.at[j]], add=True)
            pltpu.sync_copy(xbuf[b], acc2.at[idx2_v.at[j]], add=True)
            if j + 2 < kch:
                base = wid * ew + (j + 2) * CH
                rcp[j + 2] = (pltpu.async_copy(pk_hbm.at[pl.ds(base, CH)],
                                               pbuf[b], rsem[b]),
                              pltpu.async_copy(exr_hbm.at[pl.ds(base, CH)],
                                               xbuf[b], rsem[b]))
        plsc.subcore_barrier()
        pltpu.sync_copy(acc.at[pl.ds(s * rpt, rpt)],
                        out_hbm.at[c, pl.ds(s * rpt, rpt)])
        pltpu.sync_copy(acc2.at[pl.ds(s * rpt2, rpt2)],
                        out2_hbm.at[c, pl.ds(s * rpt2, rpt2)])

    return scatter_k(pk, exr, didx, didx2, zeros)


def _epi_body(npart, x_ref, *refs):
    nums = refs[:npart]
    dens = refs[npart:2 * npart]
    (pdw, pdb, ga, gx, gb, sw, sb, ow, ob,
     postw, postb, ffprew, ffpreb, ffw1, ffb1, ffw2, ffb2,
     ffpostw, ffpostb, rmat, out_ref) = refs[2 * npart:]
    x = x_ref[...]
    num = nums[0][...]
    for r in nums[1:]:
        num = num + r[...]
    den8 = dens[0][...]
    for r in dens[1:]:
        den8 = den8 + r[...]
    den = jnp.dot(den8, rmat[...], preferred_element_type=jnp.float32) + 1e-16
    agg = num / den
    xd = _ln(x, pdw[...], pdb[...])
    glog = (jnp.dot(agg, ga[...], preferred_element_type=jnp.float32)
            + jnp.dot(xd, gx[...], preferred_element_type=jnp.float32)
            + gb[...])
    g = 1.0 / (1.0 + jnp.exp(-glog))
    s = jnp.dot(xd, sw[...], preferred_element_type=jnp.float32) + sb[...]
    upd = agg + g * (s - agg)
    attn_out = jnp.dot(upd, ow[...], preferred_element_type=jnp.float32) + ob[...]
    x1 = x + _ln(attn_out, postw[...], postb[...])
    ffin = _ln(x1, ffprew[...], ffpreb[...])
    hdn = jnp.maximum(
        jnp.dot(ffin, ffw1[...], preferred_element_type=jnp.float32) + ffb1[...],
        0.0)
    h2 = jnp.dot(hdn, ffw2[...], preferred_element_type=jnp.float32) + ffb2[...]
    out_ref[...] = x1 + _ln(h2, ffpostw[...], ffpostb[...])


def _run_epilogue(x, nums, dens, p, rmat):
    N = x.shape[0]
    RB = 2000
    grid = N // RB
    npart = len(nums)
    full = lambda a: pl.BlockSpec(a.shape, lambda i: (0,) * a.ndim)
    row = lambda w: pl.BlockSpec((RB, w), lambda i: (i, 0))
    vec = lambda name: p[name].reshape(1, -1)
    args = [vec('pre_dst_w'), vec('pre_dst_b'),
            p['g_w'][:H, :], p['g_w'][H:, :], vec('g_b'),
            p['s_w'], vec('s_b'), p['o_w'], vec('o_b'),
            vec('post_w'), vec('post_b'), vec('ffpre_w'), vec('ffpre_b'),
            p['ff_w1'], vec('ff_b1'), p['ff_w2'], vec('ff_b2'),
            vec('ffpost_w'), vec('ffpost_b'), rmat]
    return pl.pallas_call(
        functools.partial(_epi_body, npart),
        grid=(grid,),
        in_specs=[row(H)] + [row(H)] * npart + [row(8)] * npart +
                 [full(a) for a in args],
        out_specs=row(H),
        out_shape=jax.ShapeDtypeStruct((N, H), jnp.float32),
    )(x, *nums, *dens, *args)


def _edge_weight_pack(p):
    w1 = jnp.stack([p['mlp%d_w1' % i] for i in range(IN_DIM)])
    pack = [
        jnp.stack([p['freq_%d' % i] for i in range(IN_DIM)]),
        w1[:, 0:NF, :],
        w1[:, NF:2 * NF, :],
        w1[:, 2 * NF, :],
        jnp.stack([p['mlp%d_b1' % i] for i in range(IN_DIM)]),
        jnp.stack([p['mlp%d_lnw' % i] for i in range(IN_DIM)]),
        jnp.stack([p['mlp%d_lnb' % i] for i in range(IN_DIM)]),
        jnp.stack([p['mlp%d_w2' % i] for i in range(IN_DIM)]),
        jnp.stack([p['mlp%d_b2' % i] for i in range(IN_DIM)]),
        p['fe_out_lnw'].reshape(1, H),
        p['fe_out_lnb'].reshape(1, H),
        p['fe_out_w'],
        p['fe_out_b'].reshape(1, H),
        p['pre_r_w'].reshape(1, H),
        p['pre_r_b'].reshape(1, H),
        p['kr_w'],
        p['vr_w'],
        p['vr_b'].reshape(1, H),
    ]
    hid = jnp.arange(H, dtype=jnp.int32) // HD
    smat = (hid[:, None] == jnp.arange(NH)[None, :]).astype(jnp.float32)
    rmat = smat.T
    lane = jnp.arange(H, dtype=jnp.int32)
    e1g = (jnp.arange(16)[:, None] == (lane // 8)[None, :]).astype(jnp.float32)
    e2f = (jnp.arange(NH)[:, None] == (lane % 8)[None, :]).astype(jnp.float32)
    pack += [smat, rmat, e1g, e2f]
    return pack, rmat


def kernel(src_x, src_pos, src_head, edges, params):
    A, M, T, C = src_x.shape
    N = A * M * T
    E = edges.shape[1]
    x = src_x.reshape(N, C)
    pos = src_pos.reshape(N, 2)
    head = src_head.reshape(N, 1)
    edges = edges.astype(jnp.int32)
    kch = E // (NW * CH)
    nch = 4
    splits = [(kch + i) // nch for i in range(nch)]
    bounds = [0]
    for s in splits:
        bounds.append(bounds[-1] + NW * CH * s)

    srct, dstt, t3 = _run_prologue(T, x, pos, head, params)
    wpack, rmat = _edge_weight_pack(params)
    zeros = jnp.zeros((NPAD, H), jnp.float32)
    nums, dens = [], []
    for lo, hi in zip(bounds[:-1], bounds[1:]):
        kc = (hi - lo) // (NW * CH)
        sidx = edges[0, lo:hi].reshape(NW, kc, CH)
        didx = edges[1, lo:hi].reshape(NW, kc, CH)
        sg, dg = _run_gather(srct, dstt, sidx, didx)
        pk, exr = _run_edges(sg, dg, t3, float(T - 1), wpack)
        part, part2 = _run_scatter(pk, exr, didx, didx // 16, zeros)
        den = part2.reshape(2, NPAD, 8)
        nums += [part[0, :N], part[1, :N]]
        dens += [den[0, :N], den[1, :N]]
    out = _run_epilogue(x, nums, dens, params, rmat)
    return out.reshape(A, M, T, C)
